# baseline (speedup 1.0000x reference)
import functools

import jax
import jax.numpy as jnp
from jax.experimental import pallas as pl
from jax.experimental.pallas import tpu as pltpu
from jax.experimental.pallas import tpu_sc as plsc

_LANES = 16
_CHUNK = 4096


def kernel(x, add_table, mult_table):
    B, L, D = x.shape
    xf = x.reshape(B, L * D)
    addf = add_table.reshape(L * D)
    multf = mult_table.reshape(L * D)
    mesh = plsc.VectorSubcoreMesh(core_axis_name="core",
                                  subcore_axis_name="subcore")

    @functools.partial(
        pl.kernel,
        out_type=jax.ShapeDtypeStruct((B, L * D), x.dtype),
        mesh=mesh,
    )
    def run(x_hbm, add_hbm, mult_hbm, o_hbm):
        def body(x_v, add_v, mult_v, o_v):
            @pl.loop(0, _CHUNK, step=_LANES, unroll=4)
            def _col(c):
                sl = pl.ds(c, _LANES)
                a = add_v[sl]
                m = mult_v[sl]
                for b in range(B):
                    o_v[b, sl] = x_v[b, sl] * m + a

        pltpu.emit_pipeline(
            body,
            grid=(L * D // _CHUNK,),
            in_specs=[
                pl.BlockSpec((B, _CHUNK), lambda i: (0, i)),
                pl.BlockSpec((_CHUNK,), lambda i: (i,)),
                pl.BlockSpec((_CHUNK,), lambda i: (i,)),
            ],
            out_specs=[pl.BlockSpec((B, _CHUNK), lambda i: (0, i))],
            core_axis_name=("core", "subcore"),
            dimension_semantics=(pltpu.PARALLEL,),
        )(x_hbm, add_hbm, mult_hbm, o_hbm)

    return run(xf, addf, multf).reshape(B, L, D)


# speedup vs baseline: 1.6214x; 1.6214x over previous
import functools

import jax
import jax.numpy as jnp
from jax.experimental import pallas as pl
from jax.experimental.pallas import tpu as pltpu
from jax.experimental.pallas import tpu_sc as plsc

_LANES = 16
_CHUNK = 4096


_BR = 4


def kernel(x, add_table, mult_table):
    B, L, D = x.shape
    mesh = plsc.VectorSubcoreMesh(core_axis_name="core",
                                  subcore_axis_name="subcore")

    @functools.partial(
        pl.kernel,
        out_type=jax.ShapeDtypeStruct((B, L, D), x.dtype),
        mesh=mesh,
    )
    def run(x_hbm, add_hbm, mult_hbm, o_hbm):
        def body(x_v, add_v, mult_v, o_v):
            @pl.loop(0, _BR)
            def _row(r):
                @pl.loop(0, D, step=_LANES, unroll=4)
                def _col(c):
                    sl = pl.ds(c, _LANES)
                    a = add_v[r, sl]
                    m = mult_v[r, sl]
                    for b in range(B):
                        o_v[b, r, sl] = x_v[b, r, sl] * m + a

        pltpu.emit_pipeline(
            body,
            grid=(L // _BR,),
            in_specs=[
                pl.BlockSpec((B, _BR, D), lambda i: (0, i, 0)),
                pl.BlockSpec((_BR, D), lambda i: (i, 0)),
                pl.BlockSpec((_BR, D), lambda i: (i, 0)),
            ],
            out_specs=[pl.BlockSpec((B, _BR, D), lambda i: (0, i, 0))],
            core_axis_name=("core", "subcore"),
            dimension_semantics=(pltpu.PARALLEL,),
        )(x_hbm, add_hbm, mult_hbm, o_hbm)

    return run(x, add_table, mult_table)


# speedup vs baseline: 5.0802x; 3.1333x over previous
import functools

import jax
import jax.numpy as jnp
from jax.experimental import pallas as pl
from jax.experimental.pallas import tpu as pltpu

_BL = 256


def kernel(x, add_table, mult_table):
    B, L, D = x.shape

    def body(x_ref, add_ref, mult_ref, o_ref):
        o_ref[...] = x_ref[...] * mult_ref[...][None] + add_ref[...][None]

    grid = (L // _BL, B)
    return pl.pallas_call(
        body,
        grid=grid,
        in_specs=[
            pl.BlockSpec((1, _BL, D), lambda i, b: (b, i, 0)),
            pl.BlockSpec((_BL, D), lambda i, b: (i, 0)),
            pl.BlockSpec((_BL, D), lambda i, b: (i, 0)),
        ],
        out_specs=pl.BlockSpec((1, _BL, D), lambda i, b: (b, i, 0)),
        out_shape=jax.ShapeDtypeStruct((B, L, D), x.dtype),
        compiler_params=pltpu.CompilerParams(
            dimension_semantics=("arbitrary", "arbitrary"),
        ),
    )(x, add_table, mult_table)
